# baseline (speedup 1.0000x reference)
import functools

import jax
import jax.numpy as jnp
from jax import lax
from jax.experimental import pallas as pl
from jax.experimental.pallas import tpu as pltpu
from jax.experimental.pallas import tpu_sc as plsc

_NC = 2
_NS = 16
_NW = _NC * _NS
_LANES = 16
_CHUNK = 128


def _make_kernel(vocab, dim, batch):
    b_per_w = batch // _NW
    n_chunks = b_per_w // _CHUNK
    n_groups = b_per_w // _LANES
    mesh = plsc.VectorSubcoreMesh(core_axis_name="c", subcore_axis_name="s")

    @functools.partial(
        pl.kernel,
        out_type=jax.ShapeDtypeStruct((batch,), jnp.float32),
        mesh=mesh,
        compiler_params=pltpu.CompilerParams(
            needs_layout_passes=False, use_tc_tiling_on_sc=False),
        scratch_types=[
            pltpu.VMEM((n_chunks, _CHUNK), jnp.int32),
            pltpu.VMEM((n_chunks, _CHUNK), jnp.int32),
            pltpu.VMEM((b_per_w, dim), jnp.float32),
            pltpu.VMEM((b_per_w, dim), jnp.float32),
            pltpu.VMEM((b_per_w,), jnp.float32),
            pltpu.SemaphoreType.DMA,
        ],
    )
    def k(hero_ids, ctx_ids, hero_tab, ctx_tab, out,
          hidx_v, cidx_v, hrow_v, crow_v, score_v, sem):
        wid = lax.axis_index("s") * _NC + lax.axis_index("c")
        base = wid * b_per_w

        pltpu.sync_copy(hero_ids.at[wid], hidx_v)
        pltpu.sync_copy(ctx_ids.at[wid], cidx_v)

        copies = []
        for j in range(n_chunks):
            rows = pl.ds(j * _CHUNK, _CHUNK)
            copies.append(
                pltpu.async_copy(hero_tab.at[hidx_v.at[j]], hrow_v.at[rows], sem))
            copies.append(
                pltpu.async_copy(ctx_tab.at[cidx_v.at[j]], crow_v.at[rows], sem))
        for c in copies:
            c.wait()

        lane = lax.iota(jnp.int32, _LANES)

        def group(g, carry):
            rows = g * _LANES + lane
            acc = jnp.zeros((_LANES,), jnp.float32)
            for d in range(dim):
                col = jnp.full((_LANES,), d, jnp.int32)
                h = plsc.load_gather(hrow_v, [rows, col])
                c = plsc.load_gather(crow_v, [rows, col])
                acc = acc + h * c
            score_v[pl.ds(g * _LANES, _LANES)] = acc
            return carry

        lax.fori_loop(0, n_groups, group, 0)

        pltpu.sync_copy(score_v, out.at[pl.ds(base, b_per_w)])

    return k


@jax.jit
def kernel(hero_ids, context_ids, hero_table, context_table):
    vocab, dim = hero_table.shape
    batch = hero_ids.shape[0]
    b_per_w = batch // _NW
    n_chunks = b_per_w // _CHUNK
    k = _make_kernel(vocab, dim, batch)
    hero3 = hero_ids.astype(jnp.int32).reshape(_NW, n_chunks, _CHUNK)
    ctx3 = context_ids.astype(jnp.int32).reshape(_NW, n_chunks, _CHUNK)
    return k(hero3, ctx3, hero_table, context_table)


# speedup vs baseline: 2.1919x; 2.1919x over previous
import functools

import jax
import jax.numpy as jnp
from jax import lax
from jax.experimental import pallas as pl
from jax.experimental.pallas import tpu as pltpu
from jax.experimental.pallas import tpu_sc as plsc

_NC = 2
_NS = 16
_NW = _NC * _NS
_LANES = 16


def _make_kernel(vocab, dim, batch):
    b_per_w = batch // _NW
    n_groups = b_per_w // _LANES
    mesh = plsc.VectorSubcoreMesh(core_axis_name="c", subcore_axis_name="s")

    @functools.partial(
        pl.kernel,
        out_type=jax.ShapeDtypeStruct((batch,), jnp.float32),
        mesh=mesh,
        compiler_params=pltpu.CompilerParams(needs_layout_passes=False),
        scratch_types=[
            pltpu.VMEM((b_per_w,), jnp.int32),
            pltpu.VMEM((b_per_w,), jnp.int32),
            pltpu.VMEM((_LANES, 8, dim), jnp.float32),
            pltpu.VMEM((_LANES, 8, dim), jnp.float32),
            pltpu.VMEM((b_per_w,), jnp.float32),
            pltpu.SemaphoreType.DMA,
        ],
    )
    def k(hero_ids, ctx_ids, hero_tab, ctx_tab, out,
          hidx_v, cidx_v, hbuf, cbuf, score_v, sem):
        wid = lax.axis_index("s") * _NC + lax.axis_index("c")
        base = wid * b_per_w

        pltpu.sync_copy(hero_ids.at[pl.ds(base, b_per_w)], hidx_v)
        pltpu.sync_copy(ctx_ids.at[pl.ds(base, b_per_w)], cidx_v)

        lane = lax.iota(jnp.int32, _LANES)

        def group(g, carry):
            e0 = g * _LANES
            hiv = hidx_v[pl.ds(e0, _LANES)]
            civ = cidx_v[pl.ds(e0, _LANES)]
            htile = lax.shift_right_logical(hiv, 3)
            ctile = lax.shift_right_logical(civ, 3)
            copies = []
            for j in range(_LANES):
                copies.append(pltpu.async_copy(
                    hero_tab.at[htile[j]], hbuf.at[j], sem))
                copies.append(pltpu.async_copy(
                    ctx_tab.at[ctile[j]], cbuf.at[j], sem))
            for c in copies:
                c.wait()

            hsub = hiv & 7
            csub = civ & 7
            acc = jnp.zeros((_LANES,), jnp.float32)
            for d in range(dim):
                col = jnp.full((_LANES,), d, jnp.int32)
                h = plsc.load_gather(hbuf, [lane, hsub, col])
                c = plsc.load_gather(cbuf, [lane, csub, col])
                acc = acc + h * c
            score_v[pl.ds(e0, _LANES)] = acc
            return carry

        lax.fori_loop(0, n_groups, group, 0)

        pltpu.sync_copy(score_v, out.at[pl.ds(base, b_per_w)])

    return k


@jax.jit
def kernel(hero_ids, context_ids, hero_table, context_table):
    vocab, dim = hero_table.shape
    batch = hero_ids.shape[0]
    k = _make_kernel(vocab, dim, batch)
    hero3 = hero_table.reshape(vocab // 8, 8, dim)
    ctx3 = context_table.reshape(vocab // 8, 8, dim)
    return k(hero_ids.astype(jnp.int32), context_ids.astype(jnp.int32),
             hero3, ctx3)
